# Initial kernel scaffold; baseline (speedup 1.0000x reference)
import jax
import jax.numpy as jnp
from jax.experimental import pallas as pl


def kernel(centers_i, centers_j, types_i, types_j, rel, tt):
    raise NotImplementedError("write your pallas kernel here")


# speedup vs baseline: 316.2492x; 316.2492x over previous
import jax
import jax.numpy as jnp
from jax.experimental import pallas as pl
from jax.experimental.pallas import tpu as pltpu

MAXD = 20
NBUCKET = 2 * MAXD + 1
TPAD = 48
TI = 64


def _body(ci_ref, cj_ref, tif_ref, tjf_ref, relT_ref, coef_ref, out_ref):
    ci = ci_ref[0]
    cj = cj_ref[0]
    dist = ci - cj
    idx = jnp.round(jnp.clip(dist, -float(MAXD), float(MAXD)) + float(MAXD))
    idx = idx.astype(jnp.int32)
    tif = tif_ref[0]
    tjf = tjf_ref[0]
    n_heads = relT_ref.shape[0]
    ti = ci.shape[0]
    for h in range(n_heads):
        tab = jnp.broadcast_to(relT_ref[h, :][None, :], (ti, TPAD))
        g = jnp.take_along_axis(tab, idx, axis=1, mode="promise_in_bounds")
        a = coef_ref[h, 0]
        b = coef_ref[h, 1]
        c = coef_ref[h, 2]
        d = coef_ref[h, 3]
        u = a + b * tif
        v = c + d * tif
        out_ref[0, h] = g + u + v * tjf


def kernel(centers_i, centers_j, types_i, types_j, rel, tt):
    B, N = centers_i.shape
    H = rel.shape[1]

    ci3 = centers_i.astype(jnp.float32).reshape(B, N, 1)
    cj3 = centers_j.astype(jnp.float32).reshape(B, 1, N)
    tif3 = types_i.astype(jnp.float32).reshape(B, N, 1)
    tjf3 = types_j.astype(jnp.float32).reshape(B, 1, N)
    relT = jnp.pad(rel, ((0, TPAD - NBUCKET), (0, 0))).T
    a = tt[:, 0, 0]
    b = tt[:, 1, 0] - tt[:, 0, 0]
    c = tt[:, 0, 1] - tt[:, 0, 0]
    d = tt[:, 1, 1] - tt[:, 1, 0] - tt[:, 0, 1] + tt[:, 0, 0]
    coef = jnp.stack([a, b, c, d], axis=1).astype(jnp.float32)

    grid = (B, N // TI)
    out = pl.pallas_call(
        _body,
        grid=grid,
        in_specs=[
            pl.BlockSpec((1, TI, 1), lambda b_, g: (b_, g, 0)),
            pl.BlockSpec((1, 1, N), lambda b_, g: (b_, 0, 0)),
            pl.BlockSpec((1, TI, 1), lambda b_, g: (b_, g, 0)),
            pl.BlockSpec((1, 1, N), lambda b_, g: (b_, 0, 0)),
            pl.BlockSpec((H, TPAD), lambda b_, g: (0, 0)),
            pl.BlockSpec(memory_space=pltpu.SMEM),
        ],
        out_specs=pl.BlockSpec((1, H, TI, N), lambda b_, g: (b_, 0, g, 0)),
        out_shape=jax.ShapeDtypeStruct((B, H, N, N), jnp.float32),
    )(ci3, cj3, tif3, tjf3, relT, coef)
    return out


# speedup vs baseline: 607.3349x; 1.9204x over previous
import jax
import jax.numpy as jnp
from jax.experimental import pallas as pl
from jax.experimental.pallas import tpu as pltpu

MAXD = 20
NBUCKET = 2 * MAXD + 1
TSTRIDE = 48
TPAD = 128
TI = 128
CJ = 2048
JSUB = 256


def _body(ci_ref, cj_ref, tii_ref, tif_ref, tjf_ref, t32_ref, coef_ref,
          out_ref):
    ci = ci_ref[0]
    cj = cj_ref[0]
    dist = ci - cj
    idx = jnp.round(jnp.clip(dist, -float(MAXD), float(MAXD)) + float(MAXD))
    cidx = idx.astype(jnp.int32) + tii_ref[0] * TSTRIDE
    tif = tif_ref[0]
    tjf = tjf_ref[0]
    n_pairs = t32_ref.shape[0]
    ti = ci.shape[0]
    himask = jnp.int32(-65536)
    wcols = []
    for h in range(2 * n_pairs):
        w0 = coef_ref[h, 0]
        w1 = coef_ref[h, 1]
        wcols.append(w0 + (w1 - w0) * tif)
    for jc in range(CJ // JSUB):
        j0 = jc * JSUB
        cidx_c = jax.lax.slice(cidx, (0, j0), (ti, j0 + JSUB))
        tjf_c = jax.lax.slice(tjf, (0, j0), (1, j0 + JSUB))
        for p in range(n_pairs):
            tab = jnp.broadcast_to(t32_ref[p, :][None, :], (ti, TPAD))
            g32 = jnp.take_along_axis(tab, cidx_c, axis=1,
                                      mode="promise_in_bounds")
            vlo = jax.lax.bitcast_convert_type(g32 << 16, jnp.float32)
            vhi = jax.lax.bitcast_convert_type(g32 & himask, jnp.float32)
            for s, v in ((0, vlo), (1, vhi)):
                h = 2 * p + s
                out_ref[0, h, :, j0:j0 + JSUB] = v + wcols[h] * tjf_c


def kernel(centers_i, centers_j, types_i, types_j, rel, tt):
    B, N = centers_i.shape
    H = rel.shape[1]
    P = H // 2

    ci3 = centers_i.astype(jnp.float32).reshape(B, N, 1)
    cj3 = centers_j.astype(jnp.float32).reshape(B, 1, N)
    tii3 = types_i.astype(jnp.int32).reshape(B, N, 1)
    tif3 = types_i.astype(jnp.float32).reshape(B, N, 1)
    tjf3 = types_j.astype(jnp.float32).reshape(B, 1, N)

    base = rel.T[:, None, :] + tt[:, :, 0:1]
    base = jnp.pad(base, ((0, 0), (0, 0), (0, TSTRIDE - NBUCKET)))
    base = base.reshape(H, 2 * TSTRIDE)
    base = jnp.pad(base, ((0, 0), (0, TPAD - 2 * TSTRIDE)))
    u16 = jax.lax.bitcast_convert_type(base.astype(jnp.bfloat16), jnp.uint16)
    t32 = (u16[1::2].astype(jnp.uint32) << 16) | u16[0::2].astype(jnp.uint32)
    t32 = t32.astype(jnp.int32)
    coef = (tt[:, :, 1] - tt[:, :, 0]).astype(jnp.float32)

    grid = (B, N // TI, N // CJ)
    out = pl.pallas_call(
        _body,
        grid=grid,
        in_specs=[
            pl.BlockSpec((1, TI, 1), lambda b_, g, j: (b_, g, 0)),
            pl.BlockSpec((1, 1, CJ), lambda b_, g, j: (b_, 0, j)),
            pl.BlockSpec((1, TI, 1), lambda b_, g, j: (b_, g, 0)),
            pl.BlockSpec((1, TI, 1), lambda b_, g, j: (b_, g, 0)),
            pl.BlockSpec((1, 1, CJ), lambda b_, g, j: (b_, 0, j)),
            pl.BlockSpec((P, TPAD), lambda b_, g, j: (0, 0)),
            pl.BlockSpec(memory_space=pltpu.SMEM),
        ],
        out_specs=pl.BlockSpec((1, H, TI, CJ), lambda b_, g, j: (b_, 0, g, j)),
        out_shape=jax.ShapeDtypeStruct((B, H, N, N), jnp.float32),
    )(ci3, cj3, tii3, tif3, tjf3, t32, coef)
    return out
